# baseline (speedup 1.0000x reference)
import functools

import jax
import jax.numpy as jnp
from jax import lax
from jax.experimental import pallas as pl
from jax.experimental.pallas import tpu as pltpu
from jax.experimental.pallas import tpu_sc as plsc

N = 131072
C = 64
NBATCH = 1024
SPA = 900
NC, NS = 2, 16
NW = NC * NS
APW = N // NW
HALF = N // NC
PAD = 192
STARTS_W = 1040
CH = 128
TILE = C * SPA
BPW = NBATCH // NW

_mesh = plsc.VectorSubcoreMesh(core_axis_name="c", subcore_axis_name="s")


@functools.partial(
    pl.kernel,
    out_type=(
        jax.ShapeDtypeStruct((N + PAD,), jnp.int32),
        jax.ShapeDtypeStruct((NC * STARTS_W,), jnp.int32),
    ),
    mesh=_mesh,
    scratch_types=[
        pltpu.VMEM((APW,), jnp.int32),
        pltpu.VMEM((APW,), jnp.int32),
        pltpu.VMEM((NBATCH * 16,), jnp.int32),
        pltpu.VMEM((NBATCH * 16,), jnp.int32),
        pltpu.VMEM((APW // CH, CH), jnp.int32),
        pltpu.VMEM((APW // CH, CH), jnp.int32),
        pltpu.VMEM((STARTS_W,), jnp.int32),
        pltpu.VMEM_SHARED((NS * NBATCH * 16,), jnp.int32),
        pltpu.SemaphoreType.DMA,
    ],
    compiler_params=pltpu.CompilerParams(needs_layout_passes=False, use_tc_tiling_on_sc=False),
)
def _bin_kernel(coords_hbm, sorted_hbm, starts_hbm,
                coords_v, barr, hist, stage, vals, dsts, starts_v,
                shared, sem):
    c = lax.axis_index("c")
    s = lax.axis_index("s")
    lane = lax.iota(jnp.int32, 16)
    zero16 = jnp.zeros((16,), jnp.int32)
    ones16 = jnp.ones((16,), jnp.int32)
    base = c * HALF + s * APW

    pltpu.sync_copy(coords_hbm.at[pl.ds(base, APW)], coords_v)

    def zero_body(i, _):
        hist[pl.ds(i * 16, 16)] = zero16
        return 0
    lax.fori_loop(0, NBATCH, zero_body, 0)

    def bin_body(t, _):
        cv = coords_v[pl.ds(t * 16, 16)]
        b = cv // SPA
        sloc = cv - b * SPA
        barr[pl.ds(t * 16, 16)] = b
        r = t // 8
        q = t - r * 8
        vals[r, pl.ds(q * 16, 16)] = ((base + t * 16 + lane) << 10) + sloc
        plsc.addupdate_scatter(hist, [b * 16 + lane], ones16)
        return 0
    lax.fori_loop(0, APW // 16, bin_body, 0)

    pltpu.sync_copy(hist, shared.at[pl.ds(s * NBATCH * 16, NBATCH * 16)])
    plsc.subcore_barrier()

    def chunk_body(k, sc_start):
        for w2 in range(NS):
            pltpu.sync_copy(shared.at[pl.ds(w2 * NBATCH * 16 + k * 1024, 1024)],
                            stage.at[pl.ds(w2 * 1024, 1024)])

        def bkt_body(b2, carry):
            start, sb_vec = carry
            total_vec = zero16
            below_vec = zero16
            own = zero16
            for w2 in range(NS):
                hv = stage[pl.ds(w2 * 1024 + b2 * 16, 16)]
                total_vec = total_vec + hv
                below_vec = below_vec + jnp.where(w2 < s, hv, zero16)
                own = jnp.where(w2 == s, hv, own)
            total = jnp.sum(total_vec)
            below = jnp.sum(below_vec)
            ex = plsc.cumsum(own) - own
            boff = k * 64 + b2
            hist[pl.ds(boff * 16, 16)] = start + below + ex
            sb_vec = jnp.where(lane == (b2 % 16), start, sb_vec)

            @pl.when(b2 % 16 == 15)
            def _():
                starts_v[pl.ds((boff // 16) * 16, 16)] = sb_vec
            return (start + total, sb_vec)

        out = lax.fori_loop(0, 64, bkt_body, (sc_start, zero16))
        return out[0]

    sc_total = lax.fori_loop(0, NBATCH // 64, chunk_body, jnp.int32(0))
    starts_v[pl.ds(NBATCH, 16)] = jnp.where(lane == 0, sc_total, zero16)

    @pl.when(s == 0)
    def _():
        pltpu.sync_copy(starts_v, starts_hbm.at[pl.ds(c * STARTS_W, STARTS_W)])

    def perm_body(t, _):
        b = barr[pl.ds(t * 16, 16)]
        idx = b * 16 + lane
        dst = plsc.load_gather(hist, [idx])
        plsc.store_scatter(hist, [idx], dst + 1)
        r = t // 8
        q = t - r * 8
        dsts[r, pl.ds(q * 16, 16)] = dst + c * HALF
        return 0
    lax.fori_loop(0, APW // 16, perm_body, 0)

    def dma_body(j, _):
        pltpu.async_copy(vals.at[j], sorted_hbm.at[dsts.at[j]], sem).wait()
        return 0
    lax.fori_loop(0, APW // CH, dma_body, 0)


M = NBATCH * SPA
ZR = 450


@functools.partial(
    pl.kernel,
    out_type=jax.ShapeDtypeStruct((M + 128, C), jnp.float32),
    mesh=_mesh,
    scratch_types=[
        pltpu.VMEM((ZR, C), jnp.float32),
        pltpu.VMEM((1024, C), jnp.float32),
        pltpu.VMEM((8, CH), jnp.int32),
        pltpu.VMEM((1040,), jnp.int32),
        pltpu.VMEM((CH,), jnp.int32),
        pltpu.VMEM((CH,), jnp.int32),
        pltpu.VMEM((CH, C), jnp.float32),
        pltpu.VMEM((NC * STARTS_W + 16,), jnp.int32),
        pltpu.SemaphoreType.DMA,
        pltpu.SemaphoreType.DMA,
        pltpu.SemaphoreType.DMA,
    ],
    compiler_params=pltpu.CompilerParams(needs_layout_passes=False, use_tc_tiling_on_sc=False),
)
def _acc_kernel(enc_hbm, sorted_hbm, starts_hbm, out_hbm,
                zbuf, compact, clist, tmark, ent_v, ids_v, rows_v, st_all,
                semz, semg, sems):
    c = lax.axis_index("c")
    s = lax.axis_index("s")
    lane = lax.iota(jnp.int32, 16)
    wg = c * NS + s
    zero16f = jnp.zeros((16,), jnp.float32)
    zero16 = jnp.zeros((16,), jnp.int32)
    trash16 = M + wg * 4 + jnp.bitwise_and(lane, 3)

    def zb_body(i, _):
        zbuf[i, pl.ds(0, 16)] = zero16f
        zbuf[i, pl.ds(16, 16)] = zero16f
        zbuf[i, pl.ds(32, 16)] = zero16f
        zbuf[i, pl.ds(48, 16)] = zero16f
        return 0
    lax.fori_loop(0, ZR, zb_body, 0)

    def tm_body(i, _):
        tmark[pl.ds(i * 16, 16)] = zero16
        return 0
    lax.fori_loop(0, 1040 // 16, tm_body, 0)

    pltpu.sync_copy(starts_hbm, st_all.at[pl.ds(0, NC * STARTS_W)])

    def ext(vec, l):
        return jnp.sum(jnp.where(lane == l, vec, 0))

    def fire_zero(j):
        b = wg * BPW + j
        pltpu.make_async_copy(
            zbuf, out_hbm.at[pl.ds(b * SPA, ZR)], semz).start()
        pltpu.make_async_copy(
            zbuf, out_hbm.at[pl.ds(b * SPA + ZR, ZR)], semz).start()

    def drain_zero(j):
        b = wg * BPW + j
        pltpu.make_async_copy(
            zbuf, out_hbm.at[pl.ds(b * SPA, ZR)], semz).wait()
        pltpu.make_async_copy(
            zbuf, out_hbm.at[pl.ds(b * SPA + ZR, ZR)], semz).wait()

    fire_zero(0)

    def bucket_body(j, _):
        b = wg * BPW + j

        def tr_body(i, _):
            r2 = i // 8
            q2 = i - r2 * 8
            clist[r2, pl.ds(q2 * 16, 16)] = trash16
            return 0
        lax.fori_loop(0, 64, tr_body, 0)

        def seg(core):
            o = core * STARTS_W + (b // 16) * 16
            v1 = st_all[pl.ds(o, 16)]
            v2 = st_all[pl.ds(o + 16, 16)]
            r0 = b % 16
            start_c = ext(v1, r0)
            end_c = jnp.where(r0 == 15, ext(v2, 0), ext(v1, r0 + 1))
            return start_c, end_c

        def core_scan(core, K):
            start_c, end_c = seg(core)
            astart = (start_c // 8) * 8
            nch = (end_c - astart + CH - 1) // CH
            gbase = core * HALF + astart

            def ch_body(ch, K):
                pltpu.sync_copy(
                    sorted_hbm.at[pl.ds(gbase + ch * CH, CH)], ent_v)

                def id_body(t, _):
                    ev = ent_v[pl.ds(t * 16, 16)]
                    idv = lax.shift_right_logical(ev, 10)
                    idv = jnp.minimum(jnp.maximum(idv, 0), N - 1)
                    ids_v[pl.ds(t * 16, 16)] = idv
                    return 0
                lax.fori_loop(0, CH // 16, id_body, 0)
                pltpu.async_copy(enc_hbm.at[ids_v], rows_v, semg).wait()

                def grp_body(t, K):
                    ev = ent_v[pl.ds(t * 16, 16)]
                    sloc = jnp.bitwise_and(ev, 1023)
                    pos0 = astart + ch * CH + t * 16
                    for l in range(16):
                        valid = jnp.logical_and(pos0 + l >= start_c,
                                                pos0 + l < end_c)
                        sl = ext(sloc, l)
                        tv = ext(tmark[pl.ds(sl, 16)], 0)
                        seen = lax.shift_right_logical(tv, 10) == (j + 1)
                        first = jnp.logical_and(valid,
                                                jnp.logical_not(seen))
                        upd = jnp.logical_and(valid, seen)
                        k_this = jnp.where(seen, jnp.bitwise_and(tv, 1023), K)
                        jr = t * 16 + l

                        @pl.when(first)
                        def _():
                            tag = ((j + 1) << 10) + K
                            plsc.store_scatter(
                                tmark, [jnp.full((16,), sl, jnp.int32)],
                                jnp.full((16,), tag, jnp.int32))
                            plsc.store_scatter(
                                clist,
                                [jnp.full((16,), K // CH, jnp.int32),
                                 jnp.full((16,), K % CH, jnp.int32)],
                                jnp.full((16,), b * SPA + sl, jnp.int32))
                            for k4 in range(4):
                                compact[k_this, pl.ds(k4 * 16, 16)] = (
                                    rows_v[jr, pl.ds(k4 * 16, 16)])

                        @pl.when(upd)
                        def _():
                            for k4 in range(4):
                                cur = compact[k_this, pl.ds(k4 * 16, 16)]
                                env = rows_v[jr, pl.ds(k4 * 16, 16)]
                                compact[k_this, pl.ds(k4 * 16, 16)] = (
                                    jnp.maximum(cur, env))

                        K = K + first.astype(jnp.int32)
                    return K
                return lax.fori_loop(0, CH // 16, grp_body, K)

            return lax.fori_loop(0, nch, ch_body, K)

        K = core_scan(0, jnp.int32(0))
        K = core_scan(1, K)

        drain_zero(j)

        @pl.when(j + 1 < BPW)
        def _():
            fire_zero(j + 1)

        nsc = (K + CH - 1) // CH

        def sc_body(i, _):
            pltpu.async_copy(compact.at[pl.ds(i * CH, CH)],
                             out_hbm.at[clist.at[i]], sems).wait()
            return 0
        lax.fori_loop(0, nsc, sc_body, 0)
        return 0

    lax.fori_loop(0, BPW, bucket_body, 0)


def kernel(batch_size, agent_encodings, encode_coordinates):
    del batch_size
    sorted_packed, starts = _bin_kernel(encode_coordinates)
    out2d = _acc_kernel(agent_encodings, sorted_packed, starts)
    return (out2d[:M].reshape(NBATCH, 30, 30, C)
            .transpose(0, 3, 1, 2))


# speedup vs baseline: 1.1802x; 1.1802x over previous
import functools

import jax
import jax.numpy as jnp
from jax import lax
from jax.experimental import pallas as pl
from jax.experimental.pallas import tpu as pltpu
from jax.experimental.pallas import tpu_sc as plsc

N = 131072
C = 64
NBATCH = 1024
SPA = 900
NC, NS = 2, 16
NW = NC * NS
APW = N // NW
HALF = N // NC
PAD = 192
STARTS_W = 1040
CH = 128
TILE = C * SPA
BPW = NBATCH // NW

_mesh = plsc.VectorSubcoreMesh(core_axis_name="c", subcore_axis_name="s")


@functools.partial(
    pl.kernel,
    out_type=(
        jax.ShapeDtypeStruct((N + PAD,), jnp.int32),
        jax.ShapeDtypeStruct((NC * STARTS_W,), jnp.int32),
    ),
    mesh=_mesh,
    scratch_types=[
        pltpu.VMEM((APW,), jnp.int32),
        pltpu.VMEM((APW,), jnp.int32),
        pltpu.VMEM((NBATCH * 16,), jnp.int32),
        pltpu.VMEM((NBATCH * 16,), jnp.int32),
        pltpu.VMEM((APW // CH, CH), jnp.int32),
        pltpu.VMEM((APW // CH, CH), jnp.int32),
        pltpu.VMEM((STARTS_W,), jnp.int32),
        pltpu.VMEM_SHARED((NS * NBATCH * 16,), jnp.int32),
        pltpu.SemaphoreType.DMA,
    ],
    compiler_params=pltpu.CompilerParams(needs_layout_passes=False, use_tc_tiling_on_sc=False),
)
def _bin_kernel(coords_hbm, sorted_hbm, starts_hbm,
                coords_v, barr, hist, stage, vals, dsts, starts_v,
                shared, sem):
    c = lax.axis_index("c")
    s = lax.axis_index("s")
    lane = lax.iota(jnp.int32, 16)
    zero16 = jnp.zeros((16,), jnp.int32)
    ones16 = jnp.ones((16,), jnp.int32)
    base = c * HALF + s * APW

    pltpu.sync_copy(coords_hbm.at[pl.ds(base, APW)], coords_v)

    def zero_body(i, _):
        hist[pl.ds(i * 16, 16)] = zero16
        return 0
    lax.fori_loop(0, NBATCH, zero_body, 0)

    def bin_body(t, _):
        cv = coords_v[pl.ds(t * 16, 16)]
        b = cv // SPA
        sloc = cv - b * SPA
        barr[pl.ds(t * 16, 16)] = b
        r = t // 8
        q = t - r * 8
        vals[r, pl.ds(q * 16, 16)] = ((base + t * 16 + lane) << 10) + sloc
        plsc.addupdate_scatter(hist, [b * 16 + lane], ones16)
        return 0
    lax.fori_loop(0, APW // 16, bin_body, 0)

    pltpu.sync_copy(hist, shared.at[pl.ds(s * NBATCH * 16, NBATCH * 16)])
    plsc.subcore_barrier()

    def chunk_body(k, sc_start):
        for w2 in range(NS):
            pltpu.sync_copy(shared.at[pl.ds(w2 * NBATCH * 16 + k * 1024, 1024)],
                            stage.at[pl.ds(w2 * 1024, 1024)])

        def bkt_body(b2, carry):
            start, sb_vec = carry
            total_vec = zero16
            below_vec = zero16
            own = zero16
            for w2 in range(NS):
                hv = stage[pl.ds(w2 * 1024 + b2 * 16, 16)]
                total_vec = total_vec + hv
                below_vec = below_vec + jnp.where(w2 < s, hv, zero16)
                own = jnp.where(w2 == s, hv, own)
            total = jnp.sum(total_vec)
            below = jnp.sum(below_vec)
            ex = plsc.cumsum(own) - own
            boff = k * 64 + b2
            hist[pl.ds(boff * 16, 16)] = start + below + ex
            sb_vec = jnp.where(lane == (b2 % 16), start, sb_vec)

            @pl.when(b2 % 16 == 15)
            def _():
                starts_v[pl.ds((boff // 16) * 16, 16)] = sb_vec
            return (start + total, sb_vec)

        out = lax.fori_loop(0, 64, bkt_body, (sc_start, zero16))
        return out[0]

    sc_total = lax.fori_loop(0, NBATCH // 64, chunk_body, jnp.int32(0))
    starts_v[pl.ds(NBATCH, 16)] = jnp.where(lane == 0, sc_total, zero16)

    @pl.when(s == 0)
    def _():
        pltpu.sync_copy(starts_v, starts_hbm.at[pl.ds(c * STARTS_W, STARTS_W)])

    def perm_body(t, _):
        b = barr[pl.ds(t * 16, 16)]
        idx = b * 16 + lane
        dst = plsc.load_gather(hist, [idx])
        plsc.store_scatter(hist, [idx], dst + 1)
        r = t // 8
        q = t - r * 8
        dsts[r, pl.ds(q * 16, 16)] = dst + c * HALF
        return 0
    lax.fori_loop(0, APW // 16, perm_body, 0)

    def dma_body(j, _):
        pltpu.async_copy(vals.at[j], sorted_hbm.at[dsts.at[j]], sem).wait()
        return 0
    lax.fori_loop(0, APW // CH, dma_body, 0)


M = NBATCH * SPA


@functools.partial(
    pl.kernel,
    out_type=jax.ShapeDtypeStruct((NBATCH * TILE,), jnp.float32),
    mesh=_mesh,
    scratch_types=[
        pltpu.VMEM((TILE,), jnp.float32),
        pltpu.VMEM((TILE,), jnp.float32),
        pltpu.VMEM((1024,), jnp.int32),
        pltpu.VMEM((1024,), jnp.int32),
        pltpu.VMEM((1040,), jnp.int32),
        pltpu.VMEM((CH,), jnp.int32),
        pltpu.VMEM((CH,), jnp.int32),
        pltpu.VMEM((CH, C), jnp.float32),
        pltpu.VMEM((NC * STARTS_W + 16,), jnp.int32),
        pltpu.SemaphoreType.DMA,
        pltpu.SemaphoreType.DMA,
        pltpu.SemaphoreType.DMA,
    ],
    compiler_params=pltpu.CompilerParams(needs_layout_passes=False, use_tc_tiling_on_sc=False),
)
def _acc_kernel(enc_hbm, sorted_hbm, starts_hbm, out_hbm,
                tile_a, tile_b, clist_a, clist_b, tmark, ent_v, ids_v,
                rows_v, st_all, sem_a, sem_b, semg):
    c = lax.axis_index("c")
    s = lax.axis_index("s")
    lane = lax.iota(jnp.int32, 16)
    lane9 = lane * SPA
    wg = c * NS + s
    zero16f = jnp.zeros((16,), jnp.float32)
    zero16 = jnp.zeros((16,), jnp.int32)

    for tile in (tile_a, tile_b):
        def z_body(i, _, tile=tile):
            tile[pl.ds(i * 16, 16)] = zero16f
            return 0
        lax.fori_loop(0, TILE // 16, z_body, 0)

    def tm_body(i, _):
        tmark[pl.ds(i * 16, 16)] = zero16
        return 0
    lax.fori_loop(0, 1040 // 16, tm_body, 0)

    pltpu.sync_copy(starts_hbm, st_all.at[pl.ds(0, NC * STARTS_W)])

    def ext(vec, l):
        return jnp.sum(jnp.where(lane == l, vec, 0))

    def seg(core, b):
        o = core * STARTS_W + (b // 16) * 16
        v1 = st_all[pl.ds(o, 16)]
        v2 = st_all[pl.ds(o + 16, 16)]
        r0 = b % 16
        start_c = ext(v1, r0)
        end_c = jnp.where(r0 == 15, ext(v2, 0), ext(v1, r0 + 1))
        return start_c, end_c

    def rezero(tile, clist, k_prev):
        def v_body(v, _):
            cells = clist[pl.ds(v * 16, 16)]
            for l in range(16):
                valid = v * 16 + l < k_prev
                sl = ext(cells, l)

                @pl.when(valid)
                def _():
                    for k4 in range(4):
                        plsc.store_scatter(
                            tile, [lane9 + (k4 * 14400 + sl)], zero16f)
            return 0
        lax.fori_loop(0, (k_prev + 15) // 16, v_body, 0)

    def proc(j, tile, clist):
        b = wg * BPW + j

        def core_scan(core, K):
            start_c, end_c = seg(core, b)
            astart = (start_c // 8) * 8
            nch = (end_c - astart + CH - 1) // CH
            gbase = core * HALF + astart

            def ch_body(ch, K):
                pltpu.sync_copy(
                    sorted_hbm.at[pl.ds(gbase + ch * CH, CH)], ent_v)

                def id_body(t, _):
                    ev = ent_v[pl.ds(t * 16, 16)]
                    idv = lax.shift_right_logical(ev, 10)
                    idv = jnp.minimum(jnp.maximum(idv, 0), N - 1)
                    ids_v[pl.ds(t * 16, 16)] = idv
                    return 0
                lax.fori_loop(0, CH // 16, id_body, 0)
                pltpu.async_copy(enc_hbm.at[ids_v], rows_v, semg).wait()

                def grp_body(t, K):
                    ev = ent_v[pl.ds(t * 16, 16)]
                    sloc = jnp.bitwise_and(ev, 1023)
                    pos0 = astart + ch * CH + t * 16
                    for l in range(16):
                        valid = jnp.logical_and(pos0 + l >= start_c,
                                                pos0 + l < end_c)
                        sl = ext(sloc, l)
                        tv = ext(tmark[pl.ds(sl, 16)], 0)
                        seen = tv == (j + 1)
                        first = jnp.logical_and(valid,
                                                jnp.logical_not(seen))
                        upd = jnp.logical_and(valid, seen)
                        jr = t * 16 + l

                        @pl.when(first)
                        def _():
                            plsc.store_scatter(
                                tmark, [jnp.full((16,), sl, jnp.int32)],
                                jnp.full((16,), j + 1, jnp.int32))
                            plsc.store_scatter(
                                clist, [jnp.full((16,), K, jnp.int32)],
                                jnp.full((16,), sl, jnp.int32))
                            for k4 in range(4):
                                plsc.store_scatter(
                                    tile, [lane9 + (k4 * 14400 + sl)],
                                    rows_v[jr, pl.ds(k4 * 16, 16)])

                        @pl.when(upd)
                        def _():
                            for k4 in range(4):
                                idxv = lane9 + (k4 * 14400 + sl)
                                cur = plsc.load_gather(tile, [idxv])
                                plsc.store_scatter(
                                    tile, [idxv],
                                    jnp.maximum(
                                        cur, rows_v[jr, pl.ds(k4 * 16, 16)]))

                        K = K + first.astype(jnp.int32)
                    return K
                return lax.fori_loop(0, CH // 16, grp_body, K)

            return lax.fori_loop(0, nch, ch_body, K)

        K = core_scan(0, jnp.int32(0))
        K = core_scan(1, K)
        return b, K

    def pair_body(jj, carry):
        ka_prev, kb_prev = carry

        @pl.when(jj > 0)
        def _():
            pltpu.make_async_copy(
                tile_a, out_hbm.at[pl.ds(0, TILE)], sem_a).wait()
        rezero(tile_a, clist_a, ka_prev)
        b_a, ka = proc(2 * jj, tile_a, clist_a)
        pltpu.make_async_copy(
            tile_a, out_hbm.at[pl.ds(b_a * TILE, TILE)], sem_a).start()

        @pl.when(jj > 0)
        def _():
            pltpu.make_async_copy(
                tile_b, out_hbm.at[pl.ds(0, TILE)], sem_b).wait()
        rezero(tile_b, clist_b, kb_prev)
        b_b, kb = proc(2 * jj + 1, tile_b, clist_b)
        pltpu.make_async_copy(
            tile_b, out_hbm.at[pl.ds(b_b * TILE, TILE)], sem_b).start()
        return (ka, kb)

    lax.fori_loop(0, BPW // 2, pair_body, (jnp.int32(0), jnp.int32(0)))
    pltpu.make_async_copy(tile_a, out_hbm.at[pl.ds(0, TILE)], sem_a).wait()
    pltpu.make_async_copy(tile_b, out_hbm.at[pl.ds(0, TILE)], sem_b).wait()


def kernel(batch_size, agent_encodings, encode_coordinates):
    del batch_size
    sorted_packed, starts = _bin_kernel(encode_coordinates)
    out1d = _acc_kernel(agent_encodings, sorted_packed, starts)
    return out1d.reshape(NBATCH, C, 30, 30)


# speedup vs baseline: 1.1951x; 1.0126x over previous
import functools

import jax
import jax.numpy as jnp
from jax import lax
from jax.experimental import pallas as pl
from jax.experimental.pallas import tpu as pltpu
from jax.experimental.pallas import tpu_sc as plsc

N = 131072
C = 64
NBATCH = 1024
SPA = 900
NC, NS = 2, 16
NW = NC * NS
APW = N // NW
HALF = N // NC
PAD = 4352
STARTS_W = 1040
CH = 128
TILE = C * SPA
BPW = NBATCH // NW

_mesh = plsc.VectorSubcoreMesh(core_axis_name="c", subcore_axis_name="s")


@functools.partial(
    pl.kernel,
    out_type=(
        jax.ShapeDtypeStruct((N + PAD,), jnp.int32),
        jax.ShapeDtypeStruct((NC * STARTS_W,), jnp.int32),
    ),
    mesh=_mesh,
    scratch_types=[
        pltpu.VMEM((APW,), jnp.int32),
        pltpu.VMEM((APW,), jnp.int32),
        pltpu.VMEM((NBATCH * 16,), jnp.int32),
        pltpu.VMEM((NBATCH * 16,), jnp.int32),
        pltpu.VMEM((APW // CH, CH), jnp.int32),
        pltpu.VMEM((APW // CH, CH), jnp.int32),
        pltpu.VMEM((STARTS_W,), jnp.int32),
        pltpu.VMEM_SHARED((NS * NBATCH * 16,), jnp.int32),
        pltpu.SemaphoreType.DMA,
    ],
    compiler_params=pltpu.CompilerParams(needs_layout_passes=False, use_tc_tiling_on_sc=False),
)
def _bin_kernel(coords_hbm, sorted_hbm, starts_hbm,
                coords_v, barr, hist, stage, vals, dsts, starts_v,
                shared, sem):
    c = lax.axis_index("c")
    s = lax.axis_index("s")
    lane = lax.iota(jnp.int32, 16)
    zero16 = jnp.zeros((16,), jnp.int32)
    ones16 = jnp.ones((16,), jnp.int32)
    base = c * HALF + s * APW

    pltpu.sync_copy(coords_hbm.at[pl.ds(base, APW)], coords_v)

    def zero_body(i, _):
        hist[pl.ds(i * 16, 16)] = zero16
        return 0
    lax.fori_loop(0, NBATCH, zero_body, 0)

    def bin_body(t, _):
        cv = coords_v[pl.ds(t * 16, 16)]
        b = cv // SPA
        sloc = cv - b * SPA
        barr[pl.ds(t * 16, 16)] = b
        r = t // 8
        q = t - r * 8
        vals[r, pl.ds(q * 16, 16)] = ((base + t * 16 + lane) << 10) + sloc
        plsc.addupdate_scatter(hist, [b * 16 + lane], ones16)
        return 0
    lax.fori_loop(0, APW // 16, bin_body, 0)

    pltpu.sync_copy(hist, shared.at[pl.ds(s * NBATCH * 16, NBATCH * 16)])
    plsc.subcore_barrier()

    def chunk_body(k, sc_start):
        for w2 in range(NS):
            pltpu.sync_copy(shared.at[pl.ds(w2 * NBATCH * 16 + k * 1024, 1024)],
                            stage.at[pl.ds(w2 * 1024, 1024)])

        def bkt_body(b2, carry):
            start, sb_vec = carry
            total_vec = zero16
            below_vec = zero16
            own = zero16
            for w2 in range(NS):
                hv = stage[pl.ds(w2 * 1024 + b2 * 16, 16)]
                total_vec = total_vec + hv
                below_vec = below_vec + jnp.where(w2 < s, hv, zero16)
                own = jnp.where(w2 == s, hv, own)
            total = jnp.sum(total_vec)
            below = jnp.sum(below_vec)
            ex = plsc.cumsum(own) - own
            boff = k * 64 + b2
            hist[pl.ds(boff * 16, 16)] = start + below + ex
            sb_vec = jnp.where(lane == (b2 % 16), start, sb_vec)

            @pl.when(b2 % 16 == 15)
            def _():
                starts_v[pl.ds((boff // 16) * 16, 16)] = sb_vec
            return (start + total, sb_vec)

        out = lax.fori_loop(0, 64, bkt_body, (sc_start, zero16))
        return out[0]

    sc_total = lax.fori_loop(0, NBATCH // 64, chunk_body, jnp.int32(0))
    starts_v[pl.ds(NBATCH, 16)] = jnp.where(lane == 0, sc_total, zero16)

    @pl.when(s == 0)
    def _():
        pltpu.sync_copy(starts_v, starts_hbm.at[pl.ds(c * STARTS_W, STARTS_W)])

    def perm_body(t, _):
        b = barr[pl.ds(t * 16, 16)]
        idx = b * 16 + lane
        dst = plsc.load_gather(hist, [idx])
        plsc.store_scatter(hist, [idx], dst + 1)
        r = t // 8
        q = t - r * 8
        dsts[r, pl.ds(q * 16, 16)] = dst + c * HALF
        return 0
    lax.fori_loop(0, APW // 16, perm_body, 0)

    def dma_body(j, _):
        pltpu.make_async_copy(vals.at[j], sorted_hbm.at[dsts.at[j]], sem).start()
        return 0
    lax.fori_loop(0, APW // CH, dma_body, 0)

    def drain_body(j, _):
        pltpu.make_async_copy(vals.at[j], sorted_hbm.at[dsts.at[j]], sem).wait()
        return 0
    lax.fori_loop(0, APW // CH, drain_body, 0)


M = NBATCH * SPA
CH2 = 64
ECAP = 4224


@functools.partial(
    pl.kernel,
    out_type=jax.ShapeDtypeStruct((NBATCH * TILE,), jnp.float32),
    mesh=_mesh,
    scratch_types=[
        pltpu.VMEM((TILE,), jnp.float32),
        pltpu.VMEM((1024,), jnp.int32),
        pltpu.VMEM((1024,), jnp.int32),
        pltpu.VMEM((1040,), jnp.int32),
        pltpu.VMEM((ECAP,), jnp.int32),
        pltpu.VMEM((ECAP,), jnp.int32),
        pltpu.VMEM((2, CH2), jnp.int32),
        pltpu.VMEM((2 * CH2,), jnp.int32),
        pltpu.VMEM((2 * CH2,), jnp.int32),
        pltpu.VMEM((2 * CH2, C), jnp.float32),
        pltpu.VMEM((NC * STARTS_W + 16,), jnp.int32),
        pltpu.SemaphoreType.DMA,
        pltpu.SemaphoreType.DMA,
    ],
    compiler_params=pltpu.CompilerParams(needs_layout_passes=False, use_tc_tiling_on_sc=False),
)
def _acc_kernel(enc_hbm, sorted_hbm, starts_hbm, out_hbm,
                tile, clist, clist_p, tmark, eca, ecb, ent_fb, entloc,
                ids_v, rows_v, st_all, semo, semg):
    c = lax.axis_index("c")
    s = lax.axis_index("s")
    lane = lax.iota(jnp.int32, 16)
    lane9 = lane * SPA
    wg = c * NS + s
    zero16f = jnp.zeros((16,), jnp.float32)
    zero16 = jnp.zeros((16,), jnp.int32)

    def z_body(i, _):
        tile[pl.ds(i * 16, 16)] = zero16f
        return 0
    lax.fori_loop(0, TILE // 16, z_body, 0)

    def tm_body(i, _):
        tmark[pl.ds(i * 16, 16)] = zero16
        return 0
    lax.fori_loop(0, 1040 // 16, tm_body, 0)

    pltpu.sync_copy(starts_hbm, st_all.at[pl.ds(0, NC * STARTS_W)])

    def ext(vec, l):
        return jnp.sum(jnp.where(lane == l, vec, 0))

    def startof(core, b):
        o = core * STARTS_W + (b // 16) * 16
        v1 = st_all[pl.ds(o, 16)]
        return ext(v1, b % 16)

    def seg(core, b):
        o = core * STARTS_W + (b // 16) * 16
        v1 = st_all[pl.ds(o, 16)]
        v2 = st_all[pl.ds(o + 16, 16)]
        r0 = b % 16
        start_c = ext(v1, r0)
        end_c = jnp.where(r0 == 15, ext(v2, 0), ext(v1, r0 + 1))
        return start_c, end_c

    sega0 = startof(0, wg * BPW)
    segb8a = (sega0 // 8) * 8
    pltpu.sync_copy(sorted_hbm.at[pl.ds(segb8a, ECAP)], eca)
    sega1 = startof(1, wg * BPW)
    segb8b = (sega1 // 8) * 8
    pltpu.sync_copy(sorted_hbm.at[pl.ds(HALF + segb8b, ECAP)], ecb)

    segb8 = (segb8a, segb8b)
    ecache = (eca, ecb)

    def round_prep(b, r, bounds):
        for core in range(NC):
            astart = bounds[core][2]
            cb0 = astart - segb8[core] + r * CH2
            usec = cb0 <= ECAP - CH2

            @pl.when(jnp.logical_not(usec))
            def _():
                pltpu.sync_copy(
                    sorted_hbm.at[pl.ds(core * HALF + astart + r * CH2, CH2)],
                    ent_fb.at[core])

            for t in range(CH2 // 16):
                off = jnp.minimum(cb0 + t * 16, ECAP - 16)
                cvec = ecache[core][pl.ds(off, 16)]
                fvec = ent_fb[core, pl.ds(t * 16, 16)]
                ev = jnp.where(usec, cvec, fvec)
                entloc[pl.ds(core * CH2 + t * 16, 16)] = ev
                idv = lax.shift_right_logical(ev, 10)
                idv = jnp.minimum(jnp.maximum(idv, 0), N - 1)
                ids_v[pl.ds(core * CH2 + t * 16, 16)] = idv
        pltpu.make_async_copy(enc_hbm.at[ids_v], rows_v, semg).start()

    def round_proc(j, r, bounds, K):
        for core in range(NC):
            start_c, end_c, astart = bounds[core]

            def grp_body(t, K):
                ev = entloc[pl.ds(core * CH2 + t * 16, 16)]
                sloc = jnp.bitwise_and(ev, 1023)
                pos0 = astart + r * CH2 + t * 16
                for l in range(16):
                    valid = jnp.logical_and(pos0 + l >= start_c,
                                            pos0 + l < end_c)
                    sl = ext(sloc, l)
                    tv = ext(tmark[pl.ds(sl, 16)], 0)
                    seen = tv == (j + 1)
                    first = jnp.logical_and(valid, jnp.logical_not(seen))
                    upd = jnp.logical_and(valid, seen)
                    jr = core * CH2 + t * 16 + l

                    @pl.when(first)
                    def _():
                        plsc.store_scatter(
                            tmark, [jnp.full((16,), sl, jnp.int32)],
                            jnp.full((16,), j + 1, jnp.int32))
                        plsc.store_scatter(
                            clist, [jnp.full((16,), K, jnp.int32)],
                            jnp.full((16,), sl, jnp.int32))
                        for k4 in range(4):
                            plsc.store_scatter(
                                tile, [lane9 + (k4 * 14400 + sl)],
                                rows_v[jr, pl.ds(k4 * 16, 16)])

                    @pl.when(upd)
                    def _():
                        for k4 in range(4):
                            idxv = lane9 + (k4 * 14400 + sl)
                            cur = plsc.load_gather(tile, [idxv])
                            plsc.store_scatter(
                                tile, [idxv],
                                jnp.maximum(
                                    cur, rows_v[jr, pl.ds(k4 * 16, 16)]))

                    K = K + first.astype(jnp.int32)
                return K
            K = lax.fori_loop(0, CH2 // 16, grp_body, K)
        return K

    def rezero(k_prev):
        def v_body(v, _):
            cells = clist_p[pl.ds(v * 16, 16)]
            for l in range(16):
                valid = v * 16 + l < k_prev
                sl = ext(cells, l)

                @pl.when(valid)
                def _():
                    for k4 in range(4):
                        plsc.store_scatter(
                            tile, [lane9 + (k4 * 14400 + sl)], zero16f)
            return 0
        lax.fori_loop(0, (k_prev + 15) // 16, v_body, 0)

    def bucket_body(j, k_prev):
        b = wg * BPW + j
        bounds = []
        for core in range(NC):
            start_c, end_c = seg(core, b)
            bounds.append((start_c, end_c, (start_c // 8) * 8))
        nr0 = (bounds[0][1] - bounds[0][2] + CH2 - 1) // CH2
        nr1 = (bounds[1][1] - bounds[1][2] + CH2 - 1) // CH2
        nr = jnp.maximum(nr0, nr1)

        round_prep(b, 0, bounds)

        @pl.when(j > 0)
        def _():
            pltpu.make_async_copy(
                tile, out_hbm.at[pl.ds(0, TILE)], semo).wait()
        rezero(k_prev)
        def r_body(r, K):
            pltpu.make_async_copy(enc_hbm.at[ids_v], rows_v, semg).wait()
            K2 = round_proc(j, r, bounds, K)

            @pl.when(r + 1 < nr)
            def _():
                round_prep(b, r + 1, bounds)
            return K2
        K = lax.fori_loop(0, jnp.maximum(nr, 1), r_body, jnp.int32(0))

        def cp_body(v, _):
            clist_p[pl.ds(v * 16, 16)] = clist[pl.ds(v * 16, 16)]
            return 0
        lax.fori_loop(0, (K + 15) // 16, cp_body, 0)

        pltpu.make_async_copy(
            tile, out_hbm.at[pl.ds(b * TILE, TILE)], semo).start()
        return K

    lax.fori_loop(0, BPW, bucket_body, jnp.int32(0))
    pltpu.make_async_copy(tile, out_hbm.at[pl.ds(0, TILE)], semo).wait()


def kernel(batch_size, agent_encodings, encode_coordinates):
    del batch_size
    sorted_packed, starts = _bin_kernel(encode_coordinates)
    out1d = _acc_kernel(agent_encodings, sorted_packed, starts)
    return out1d.reshape(NBATCH, C, 30, 30)
